# baseline (speedup 1.0000x reference)
import functools

import jax
import jax.numpy as jnp
from jax import lax
from jax.experimental import pallas as pl
from jax.experimental.pallas import tpu as pltpu
from jax.experimental.pallas import tpu_sc as plsc

N = 10000
E = 320000
NPAD = 10240
EP = 327680
NG = EP // 128
G = 2
TR = NPAD // 16
ROWS_BLK = 2000

_sc_mesh = functools.partial(
    plsc.VectorSubcoreMesh, core_axis_name="c", subcore_axis_name="s"
)


@functools.partial(
    pl.kernel,
    out_type=jax.ShapeDtypeStruct((2, NPAD, 16), jnp.float32),
    mesh=_sc_mesh(),
    scratch_types=[
        pltpu.VMEM_SHARED((NPAD, 16), jnp.float32),
        pltpu.VMEM((G, 128), jnp.int32),
        pltpu.VMEM((128, 16), jnp.float32),
    ],
)
def _deg_kernel(cols_hbm, zeros16, ones16, out, acc, cidx, obuf):
    c = lax.axis_index("c")
    s = lax.axis_index("s")
    wid = c * 16 + s
    pltpu.sync_copy(ones16, obuf)
    for k in range(TR // 128):
        pltpu.sync_copy(zeros16, acc.at[pl.ds(s * TR + k * 128, 128)])
    plsc.subcore_barrier()

    tile_g = NG // 32
    g0 = wid * tile_g

    def body(i, carry):
        gb = g0 + i * G
        pltpu.sync_copy(cols_hbm.at[pl.ds(gb, G)], cidx)
        for g in range(G):
            pltpu.sync_copy(obuf, acc.at[cidx.at[g]], add=True)
        return carry

    lax.fori_loop(0, tile_g // G, body, 0)
    plsc.subcore_barrier()
    pltpu.sync_copy(acc.at[pl.ds(s * TR, TR)], out.at[c].at[pl.ds(s * TR, TR)])


GE = 64
NG64 = EP // GE
NB = 4
IG = 8


def _make_msg_kernel(edge_split):
    if edge_split:
        tile_g = NG64 // 32
        core_stride = NG64 // 2
    else:
        tile_g = NG64 // 16
        core_stride = 0
    nbatch = tile_g // IG

    @functools.partial(
        pl.kernel,
        out_type=jax.ShapeDtypeStruct((2, NPAD, 128), jnp.float32),
        mesh=_sc_mesh(),
        scratch_types=[
            pltpu.VMEM_SHARED((NPAD, 128), jnp.float32),
            pltpu.VMEM((2, IG, GE), jnp.int32),
            pltpu.VMEM((2, IG, GE), jnp.int32),
            pltpu.VMEM((NB, GE, 128), jnp.float32),
            [pltpu.SemaphoreType.DMA] * NB,
            [pltpu.SemaphoreType.DMA] * NB,
            pltpu.SemaphoreType.DMA,
        ],
    )
    def k(ylo, yhi, rows_hbm, cols_hbm, zeros128, out, acc, ridx, cidx, dbuf,
          semg, sems, semi):
        c = lax.axis_index("c")
        s = lax.axis_index("s")
        for k_ in range(TR // 128):
            pltpu.sync_copy(zeros128, acc.at[pl.ds(s * TR + k_ * 128, 128)])
        plsc.subcore_barrier()

        g0 = c * core_stride + s * tile_g

        def fire_gather(idx_slice, b):
            @pl.when(c == 0)
            def _():
                pltpu.async_copy(ylo.at[idx_slice], dbuf.at[b], semg[b])

            @pl.when(c == 1)
            def _():
                pltpu.async_copy(yhi.at[idx_slice], dbuf.at[b], semg[b])

        def wait_gather(idx_slice, b):
            pltpu.make_async_copy(ylo.at[idx_slice], dbuf.at[b], semg[b]).wait()

        def fire_scatter(b, idx_slice):
            pltpu.async_copy(dbuf.at[b], acc.at[idx_slice], sems[b], add=True)

        def wait_scatter(b, idx_slice):
            pltpu.make_async_copy(dbuf.at[b], acc.at[idx_slice], sems[b]).wait()

        def stage_idx(set_, batch):
            gb = g0 + batch * IG
            pltpu.async_copy(rows_hbm.at[pl.ds(gb, IG)], ridx.at[set_], semi)
            pltpu.async_copy(cols_hbm.at[pl.ds(gb, IG)], cidx.at[set_], semi)

        def wait_idx(set_):
            pltpu.make_async_copy(
                rows_hbm.at[pl.ds(g0, IG)], ridx.at[set_], semi).wait()
            pltpu.make_async_copy(
                cols_hbm.at[pl.ds(g0, IG)], cidx.at[set_], semi).wait()

        def batch_steps(p, first, last, stage_fn=None):
            for j in range(IG):
                b = j % NB
                wait_gather(ridx.at[p, j], b)
                fire_scatter(b, cidx.at[p, j])
                b2 = (j + 2) % NB
                if j >= 2:
                    wait_scatter(b2, cidx.at[p, j - 2])
                elif not first:
                    wait_scatter(b2, cidx.at[1 - p, IG - 2 + j])
                if j == 1 and stage_fn is not None:
                    stage_fn()
                if j == IG - 2 and not last:
                    wait_idx(1 - p)
                if j < IG - 2:
                    fire_gather(ridx.at[p, j + 2], b2)
                elif not last:
                    fire_gather(ridx.at[1 - p, j - (IG - 2)], b2)

        stage_idx(0, 0)
        wait_idx(0)
        for j in range(2):
            fire_gather(ridx.at[0, j], j)

        stage_idx(1, 1)
        batch_steps(0, first=True, last=(nbatch == 1))

        def body(kb, carry):
            p = lax.rem(kb, 2)
            batch_steps(p, first=False, last=False,
                        stage_fn=lambda: stage_idx(1 - p, kb + 1))
            return carry

        lax.fori_loop(1, nbatch - 1, body, 0)

        pf = (nbatch - 1) % 2
        batch_steps(pf, first=False, last=True)
        wait_scatter(2 % NB, cidx.at[pf, IG - 2])
        wait_scatter(3 % NB, cidx.at[pf, IG - 1])

        plsc.subcore_barrier()
        pltpu.sync_copy(
            acc.at[pl.ds(s * TR, TR)], out.at[c].at[pl.ds(s * TR, TR)]
        )

    return k


_msg_split = _make_msg_kernel(True)
_msg_halves = _make_msg_kernel(False)


def _k1_body(x_ref, d0_ref, d1_ref, w1_ref, y_ref, dis_ref):
    deg = d0_ref[...] + d1_ref[...] + 1.0
    dis = lax.rsqrt(deg)
    dis_ref[...] = dis
    y_ref[...] = jnp.dot(x_ref[...], w1_ref[...],
                         preferred_element_type=jnp.float32) * dis


def _tc_k1(x, d0, d1, W1):
    grid = (N // ROWS_BLK,)
    return pl.pallas_call(
        _k1_body,
        grid=grid,
        in_specs=[
            pl.BlockSpec((ROWS_BLK, 128), lambda r: (r, 0)),
            pl.BlockSpec((ROWS_BLK, 1), lambda r: (r, 0)),
            pl.BlockSpec((ROWS_BLK, 1), lambda r: (r, 0)),
            pl.BlockSpec((128, 128), lambda r: (0, 0)),
        ],
        out_specs=[
            pl.BlockSpec((ROWS_BLK, 128), lambda r: (r, 0)),
            pl.BlockSpec((ROWS_BLK, 1), lambda r: (r, 0)),
        ],
        out_shape=[
            jax.ShapeDtypeStruct((N, 128), jnp.float32),
            jax.ShapeDtypeStruct((N, 1), jnp.float32),
        ],
    )(x, d0, d1, W1)


def _ln(pre, g, b):
    m = jnp.mean(pre, axis=-1, keepdims=True)
    v = jnp.mean((pre - m) * (pre - m), axis=-1, keepdims=True)
    return (pre - m) * lax.rsqrt(v + 1e-5) * g + b


def _leaky(h):
    return jnp.where(h >= 0, h, 0.01 * h)


def _k2_body(a0_ref, a1_ref, y1_ref, dis_ref, b1_ref, g1_ref, bb1_ref, w2_ref,
             y2_ref):
    dis = dis_ref[...]
    pre = dis * (a0_ref[0] + a1_ref[0] + y1_ref[...]) + b1_ref[...]
    h = _leaky(_ln(pre, g1_ref[...], bb1_ref[...]))
    y2 = jnp.dot(h, w2_ref[...], preferred_element_type=jnp.float32) * dis
    y2_ref[0] = y2[:, :128]
    y2_ref[1] = y2[:, 128:]


def _tc_k2(accs, y1, dis, b1, g1, bb1, W2):
    grid = (N // ROWS_BLK,)
    return pl.pallas_call(
        _k2_body,
        grid=grid,
        in_specs=[
            pl.BlockSpec((1, ROWS_BLK, 128), lambda r: (0, r, 0)),
            pl.BlockSpec((1, ROWS_BLK, 128), lambda r: (1, r, 0)),
            pl.BlockSpec((ROWS_BLK, 128), lambda r: (r, 0)),
            pl.BlockSpec((ROWS_BLK, 1), lambda r: (r, 0)),
            pl.BlockSpec((1, 128), lambda r: (0, 0)),
            pl.BlockSpec((1, 128), lambda r: (0, 0)),
            pl.BlockSpec((1, 128), lambda r: (0, 0)),
            pl.BlockSpec((128, 256), lambda r: (0, 0)),
        ],
        out_specs=pl.BlockSpec((2, ROWS_BLK, 128), lambda r: (0, r, 0)),
        out_shape=jax.ShapeDtypeStruct((2, N, 128), jnp.float32),
    )(accs, accs, y1, dis, b1, g1, bb1, W2)


def _k3_body(alo_ref, ahi_ref, ylo_ref, yhi_ref, dis_ref, b2_ref, g2_ref,
             bb2_ref, w3_ref, y3_ref):
    dis = dis_ref[...]
    accy = jnp.concatenate(
        [alo_ref[0] + ylo_ref[0], ahi_ref[0] + yhi_ref[0]], axis=-1
    )
    pre = dis * accy + b2_ref[...]
    h = _leaky(_ln(pre, g2_ref[...], bb2_ref[...]))
    y3 = jnp.dot(h, w3_ref[...], preferred_element_type=jnp.float32) * dis
    y3_ref[0] = y3[:, :128]
    y3_ref[1] = y3[:, 128:]


def _tc_k3(accs, ycat, dis, b2, g2, bb2, W3):
    grid = (N // ROWS_BLK,)
    return pl.pallas_call(
        _k3_body,
        grid=grid,
        in_specs=[
            pl.BlockSpec((1, ROWS_BLK, 128), lambda r: (0, r, 0)),
            pl.BlockSpec((1, ROWS_BLK, 128), lambda r: (1, r, 0)),
            pl.BlockSpec((1, ROWS_BLK, 128), lambda r: (0, r, 0)),
            pl.BlockSpec((1, ROWS_BLK, 128), lambda r: (1, r, 0)),
            pl.BlockSpec((ROWS_BLK, 1), lambda r: (r, 0)),
            pl.BlockSpec((1, 256), lambda r: (0, 0)),
            pl.BlockSpec((1, 256), lambda r: (0, 0)),
            pl.BlockSpec((1, 256), lambda r: (0, 0)),
            pl.BlockSpec((256, 256), lambda r: (0, 0)),
        ],
        out_specs=pl.BlockSpec((2, ROWS_BLK, 128), lambda r: (0, r, 0)),
        out_shape=jax.ShapeDtypeStruct((2, N, 128), jnp.float32),
    )(accs, accs, ycat, ycat, dis, b2, g2, bb2, W3)


def _k4_body(alo_ref, ahi_ref, ylo_ref, yhi_ref, dis_ref, b3_ref, g3_ref,
             bb3_ref, batch_ref, bs_ref, wf_ref, bf_ref, out_ref, psum_ref):
    r = pl.program_id(0)
    nsteps = pl.num_programs(0)

    @pl.when(r == 0)
    def _():
        psum_ref[...] = jnp.zeros_like(psum_ref)

    dis = dis_ref[...]
    accy = jnp.concatenate(
        [alo_ref[0] + ylo_ref[0], ahi_ref[0] + yhi_ref[0]], axis=-1
    )
    pre = dis * accy + b3_ref[...]
    h = _ln(pre, g3_ref[...], bb3_ref[...])
    hx = jnp.concatenate([h, jnp.ones((h.shape[0], 128), jnp.float32)], axis=-1)
    cols = lax.broadcasted_iota(jnp.int32, (h.shape[0], 128), 1)
    P = (batch_ref[...] == cols).astype(jnp.float32)
    psum_ref[...] += lax.dot_general(
        P, hx, (((0,), (0,)), ((), ())), preferred_element_type=jnp.float32
    )

    @pl.when(r == nsteps - 1)
    def _():
        ps = psum_ref[...]
        cnt = jnp.maximum(ps[:, 256:257], 1.0)
        pooled = ps[:, :256] / cnt
        valid = lax.broadcasted_iota(jnp.int32, (128, 1), 0) < bs_ref[0]
        pooled = jnp.where(valid, pooled, 0.0)
        out_ref[...] = (
            jnp.dot(pooled, wf_ref[...], preferred_element_type=jnp.float32)
            + bf_ref[...]
        )


def _tc_k4(accs, ycat, dis, b3, g3, bb3, batch2d, bs, Wf, bf):
    grid = (N // ROWS_BLK,)
    return pl.pallas_call(
        _k4_body,
        grid=grid,
        in_specs=[
            pl.BlockSpec((1, ROWS_BLK, 128), lambda r: (0, r, 0)),
            pl.BlockSpec((1, ROWS_BLK, 128), lambda r: (1, r, 0)),
            pl.BlockSpec((1, ROWS_BLK, 128), lambda r: (0, r, 0)),
            pl.BlockSpec((1, ROWS_BLK, 128), lambda r: (1, r, 0)),
            pl.BlockSpec((ROWS_BLK, 1), lambda r: (r, 0)),
            pl.BlockSpec((1, 256), lambda r: (0, 0)),
            pl.BlockSpec((1, 256), lambda r: (0, 0)),
            pl.BlockSpec((1, 256), lambda r: (0, 0)),
            pl.BlockSpec((ROWS_BLK, 1), lambda r: (r, 0)),
            pl.BlockSpec(memory_space=pltpu.SMEM),
            pl.BlockSpec((256, 128), lambda r: (0, 0)),
            pl.BlockSpec((1, 128), lambda r: (0, 0)),
        ],
        out_specs=pl.BlockSpec((128, 128), lambda r: (0, 0)),
        out_shape=jax.ShapeDtypeStruct((128, 128), jnp.float32),
        scratch_shapes=[pltpu.VMEM((128, 384), jnp.float32)],
    )(accs, accs, ycat, ycat, dis, b3, g3, bb3, batch2d, bs, Wf, bf)


def kernel(x, edge_index, batch, batch_size, W1, b1, ln1_g, ln1_b, W2, b2,
           ln2_g, ln2_b, W3, b3, ln3_g, ln3_b, Wf, bf):
    row = edge_index[0].astype(jnp.int32)
    col = edge_index[1].astype(jnp.int32)
    pad = EP - E
    rows_p = jnp.concatenate([row, jnp.zeros((pad,), jnp.int32)])
    cols_p = jnp.concatenate([col, jnp.full((pad,), N, jnp.int32)])
    cols_g = cols_p.reshape(NG, 128)
    rows_g64 = rows_p.reshape(NG64, GE)
    cols_g64 = cols_p.reshape(NG64, GE)

    zeros16 = jnp.zeros((128, 16), jnp.float32)
    ones16 = jnp.ones((128, 16), jnp.float32)
    zeros128 = jnp.zeros((128, 128), jnp.float32)

    degp = _deg_kernel(cols_g, zeros16, ones16)
    d0 = degp[0, :N, 0:1]
    d1 = degp[1, :N, 0:1]
    y1, dis = _tc_k1(x, d0, d1, W1)

    acc1 = _msg_split(y1, y1, rows_g64, cols_g64, zeros128)
    y2cat = _tc_k2(acc1, y1, dis, b1.reshape(1, -1),
                   ln1_g.reshape(1, -1), ln1_b.reshape(1, -1), W2)

    acc2 = _msg_halves(y2cat[0], y2cat[1], rows_g64, cols_g64, zeros128)
    y3cat = _tc_k3(acc2, y2cat, dis, b2.reshape(1, -1),
                   ln2_g.reshape(1, -1), ln2_b.reshape(1, -1), W3)

    acc3 = _msg_halves(y3cat[0], y3cat[1], rows_g64, cols_g64, zeros128)

    bs = jnp.asarray(batch_size, jnp.int32).reshape(1)
    out = _tc_k4(acc3, y3cat, dis, b3.reshape(1, -1),
                 ln3_g.reshape(1, -1), ln3_b.reshape(1, -1),
                 batch.astype(jnp.int32).reshape(N, 1), bs, Wf,
                 bf.reshape(1, -1))
    return out


# speedup vs baseline: 1.0558x; 1.0558x over previous
import functools

import jax
import jax.numpy as jnp
from jax import lax
from jax.experimental import pallas as pl
from jax.experimental.pallas import tpu as pltpu
from jax.experimental.pallas import tpu_sc as plsc

N = 10000
E = 320000
NPAD = 10240
EP = 327680
NG = EP // 128
G = 2
TR = NPAD // 16
ROWS_BLK = 2000

_sc_mesh = functools.partial(
    plsc.VectorSubcoreMesh, core_axis_name="c", subcore_axis_name="s"
)


@functools.partial(
    pl.kernel,
    out_type=jax.ShapeDtypeStruct((2, NPAD, 16), jnp.float32),
    mesh=_sc_mesh(),
    scratch_types=[
        pltpu.VMEM_SHARED((NPAD, 16), jnp.float32),
        pltpu.VMEM((G, 128), jnp.int32),
        pltpu.VMEM((128, 16), jnp.float32),
    ],
)
def _deg_kernel(cols_hbm, zeros16, ones16, out, acc, cidx, obuf):
    c = lax.axis_index("c")
    s = lax.axis_index("s")
    wid = c * 16 + s
    pltpu.sync_copy(ones16, obuf)
    for k in range(TR // 128):
        pltpu.sync_copy(zeros16, acc.at[pl.ds(s * TR + k * 128, 128)])
    plsc.subcore_barrier()

    tile_g = NG // 32
    g0 = wid * tile_g

    def body(i, carry):
        gb = g0 + i * G
        pltpu.sync_copy(cols_hbm.at[pl.ds(gb, G)], cidx)
        for g in range(G):
            pltpu.sync_copy(obuf, acc.at[cidx.at[g]], add=True)
        return carry

    lax.fori_loop(0, tile_g // G, body, 0)
    plsc.subcore_barrier()
    pltpu.sync_copy(acc.at[pl.ds(s * TR, TR)], out.at[c].at[pl.ds(s * TR, TR)])


GE = 128
NB = 8
IG = 8
NSET = 3
TILE_G = NG // 16
NBATCH = TILE_G // IG


def _make_msg_kernel(npass):
    @functools.partial(
        pl.kernel,
        out_type=jax.ShapeDtypeStruct((2 * npass, NPAD, 64), jnp.float32),
        mesh=_sc_mesh(),
        scratch_types=[
            pltpu.VMEM_SHARED((NPAD, 64), jnp.float32),
            pltpu.VMEM((NSET, IG, GE), jnp.int32),
            pltpu.VMEM((NSET, IG, GE), jnp.int32),
            pltpu.VMEM((NB, GE, 64), jnp.float32),
            [pltpu.SemaphoreType.DMA] * NB,
            [pltpu.SemaphoreType.DMA] * NB,
            pltpu.SemaphoreType.DMA,
        ],
        compiler_params=pltpu.CompilerParams(use_tc_tiling_on_sc=False),
    )
    def k(y0, y1, y2, y3, rows_hbm, cols_hbm, zeros64, out, acc, ridx, cidx,
          dbuf, semg, sems, semi):
        c = lax.axis_index("c")
        s = lax.axis_index("s")
        ys = [y0, y1, y2, y3]
        g0 = s * TILE_G

        def stage_idx(set_, batch):
            gb = g0 + batch * IG
            pltpu.async_copy(rows_hbm.at[pl.ds(gb, IG)], ridx.at[set_], semi)
            pltpu.async_copy(cols_hbm.at[pl.ds(gb, IG)], cidx.at[set_], semi)

        def wait_idx(set_):
            pltpu.make_async_copy(
                rows_hbm.at[pl.ds(g0, IG)], ridx.at[set_], semi).wait()
            pltpu.make_async_copy(
                cols_hbm.at[pl.ds(g0, IG)], cidx.at[set_], semi).wait()

        for t in range(npass):
            src0 = ys[t]
            src1 = ys[npass + t]

            def fire_gather(idx_slice, b):
                @pl.when(c == 0)
                def _():
                    pltpu.async_copy(src0.at[idx_slice], dbuf.at[b], semg[b])

                @pl.when(c == 1)
                def _():
                    pltpu.async_copy(src1.at[idx_slice], dbuf.at[b], semg[b])

            def wait_gather(idx_slice, b):
                pltpu.make_async_copy(
                    src0.at[idx_slice], dbuf.at[b], semg[b]).wait()

            def fire_scatter(b, idx_slice):
                pltpu.async_copy(dbuf.at[b], acc.at[idx_slice], sems[b],
                                 add=True)

            def wait_scatter(b, idx_slice):
                pltpu.make_async_copy(
                    dbuf.at[b], acc.at[idx_slice], sems[b]).wait()

            for k_ in range(TR // 128):
                pltpu.sync_copy(zeros64, acc.at[pl.ds(s * TR + k_ * 128, 128)])
            plsc.subcore_barrier()

            def batch_steps(p, pn, first, last, stage_fn=None):
                for j in range(IG):
                    b = j % NB
                    wait_gather(ridx.at[p, j], b)
                    fire_scatter(b, cidx.at[p, j])
                    bprev = (j - 1) % NB
                    if j == 0:
                        if not first:
                            wait_scatter(bprev, cidx.at[p, IG - 1])
                        fire_gather(ridx.at[p, IG - 1], IG - 1)
                    else:
                        wait_scatter(bprev, cidx.at[p, j - 1])
                        if j == 1 and not last:
                            wait_idx(pn)
                        if j == 6 and stage_fn is not None:
                            stage_fn()
                        if not last:
                            fire_gather(ridx.at[pn, j - 1], bprev)

            stage_idx(0, 0)
            wait_idx(0)
            stage_idx(1, 1)
            for j in range(NB - 1):
                fire_gather(ridx.at[0, j], j)

            batch_steps(0, 1, first=True, last=False,
                        stage_fn=lambda: stage_idx(2, 2))

            def body(kb, carry):
                p = lax.rem(kb, NSET)
                pn = lax.rem(kb + 1, NSET)

                def stage():
                    @pl.when(kb + 2 < NBATCH)
                    def _():
                        stage_idx(lax.rem(kb + 2, NSET), kb + 2)

                batch_steps(p, pn, first=False, last=False, stage_fn=stage)
                return carry

            lax.fori_loop(1, NBATCH - 1, body, 0)

            pf = (NBATCH - 1) % NSET
            batch_steps(pf, 0, first=False, last=True)
            wait_scatter(IG - 1, cidx.at[pf, IG - 1])

            plsc.subcore_barrier()
            pltpu.sync_copy(
                acc.at[pl.ds(s * TR, TR)],
                out.at[c * npass + t].at[pl.ds(s * TR, TR)],
            )
            if t + 1 < npass:
                plsc.subcore_barrier()

    return k


_msg_1pass = _make_msg_kernel(1)
_msg_2pass = _make_msg_kernel(2)


def _split64(y_ref, y):
    for i in range(y_ref.shape[0]):
        y_ref[i] = y[:, 64 * i:64 * i + 64]


def _k1_body(x_ref, d0_ref, d1_ref, w1_ref, y_ref, dis_ref):
    deg = d0_ref[...] + d1_ref[...] + 1.0
    dis = lax.rsqrt(deg)
    dis_ref[...] = dis
    _split64(y_ref, jnp.dot(x_ref[...], w1_ref[...],
                            preferred_element_type=jnp.float32) * dis)


def _tc_k1(x, d0, d1, W1):
    grid = (N // ROWS_BLK,)
    return pl.pallas_call(
        _k1_body,
        grid=grid,
        in_specs=[
            pl.BlockSpec((ROWS_BLK, 128), lambda r: (r, 0)),
            pl.BlockSpec((ROWS_BLK, 1), lambda r: (r, 0)),
            pl.BlockSpec((ROWS_BLK, 1), lambda r: (r, 0)),
            pl.BlockSpec((128, 128), lambda r: (0, 0)),
        ],
        out_specs=[
            pl.BlockSpec((2, ROWS_BLK, 64), lambda r: (0, r, 0)),
            pl.BlockSpec((ROWS_BLK, 1), lambda r: (r, 0)),
        ],
        out_shape=[
            jax.ShapeDtypeStruct((2, N, 64), jnp.float32),
            jax.ShapeDtypeStruct((N, 1), jnp.float32),
        ],
    )(x, d0, d1, W1)


def _ln(pre, g, b):
    m = jnp.mean(pre, axis=-1, keepdims=True)
    v = jnp.mean((pre - m) * (pre - m), axis=-1, keepdims=True)
    return (pre - m) * lax.rsqrt(v + 1e-5) * g + b


def _leaky(h):
    return jnp.where(h >= 0, h, 0.01 * h)


def _k2_body(a0_ref, a1_ref, y1a_ref, y1b_ref, dis_ref, b1_ref, g1_ref,
             bb1_ref, w2_ref, y2_ref):
    dis = dis_ref[...]
    acc = jnp.concatenate([a0_ref[0], a1_ref[0]], axis=-1)
    y1 = jnp.concatenate([y1a_ref[0], y1b_ref[0]], axis=-1)
    pre = dis * (acc + y1) + b1_ref[...]
    h = _leaky(_ln(pre, g1_ref[...], bb1_ref[...]))
    _split64(y2_ref, jnp.dot(h, w2_ref[...],
                             preferred_element_type=jnp.float32) * dis)


def _tc_k2(accs, y1, dis, b1, g1, bb1, W2):
    grid = (N // ROWS_BLK,)
    return pl.pallas_call(
        _k2_body,
        grid=grid,
        in_specs=[
            pl.BlockSpec((1, ROWS_BLK, 64), lambda r: (0, r, 0)),
            pl.BlockSpec((1, ROWS_BLK, 64), lambda r: (1, r, 0)),
            pl.BlockSpec((1, ROWS_BLK, 64), lambda r: (0, r, 0)),
            pl.BlockSpec((1, ROWS_BLK, 64), lambda r: (1, r, 0)),
            pl.BlockSpec((ROWS_BLK, 1), lambda r: (r, 0)),
            pl.BlockSpec((1, 128), lambda r: (0, 0)),
            pl.BlockSpec((1, 128), lambda r: (0, 0)),
            pl.BlockSpec((1, 128), lambda r: (0, 0)),
            pl.BlockSpec((128, 256), lambda r: (0, 0)),
        ],
        out_specs=pl.BlockSpec((4, ROWS_BLK, 64), lambda r: (0, r, 0)),
        out_shape=jax.ShapeDtypeStruct((4, N, 64), jnp.float32),
    )(accs, accs, y1, y1, dis, b1, g1, bb1, W2)


def _chunk_specs(n):
    return [pl.BlockSpec((1, ROWS_BLK, 64), (lambda i: lambda r: (i, r, 0))(i))
            for i in range(n)]


def _k3_body(a0, a1, a2, a3, y0, y1, y2, y3, dis_ref, b2_ref, g2_ref,
             bb2_ref, w3_ref, y3_ref):
    dis = dis_ref[...]
    acc = jnp.concatenate([a0[0], a1[0], a2[0], a3[0]], axis=-1)
    y = jnp.concatenate([y0[0], y1[0], y2[0], y3[0]], axis=-1)
    pre = dis * (acc + y) + b2_ref[...]
    h = _leaky(_ln(pre, g2_ref[...], bb2_ref[...]))
    _split64(y3_ref, jnp.dot(h, w3_ref[...],
                             preferred_element_type=jnp.float32) * dis)


def _tc_k3(accs, ycat, dis, b2, g2, bb2, W3):
    grid = (N // ROWS_BLK,)
    return pl.pallas_call(
        _k3_body,
        grid=grid,
        in_specs=_chunk_specs(4) + _chunk_specs(4) + [
            pl.BlockSpec((ROWS_BLK, 1), lambda r: (r, 0)),
            pl.BlockSpec((1, 256), lambda r: (0, 0)),
            pl.BlockSpec((1, 256), lambda r: (0, 0)),
            pl.BlockSpec((1, 256), lambda r: (0, 0)),
            pl.BlockSpec((256, 256), lambda r: (0, 0)),
        ],
        out_specs=pl.BlockSpec((4, ROWS_BLK, 64), lambda r: (0, r, 0)),
        out_shape=jax.ShapeDtypeStruct((4, N, 64), jnp.float32),
    )(accs, accs, accs, accs, ycat, ycat, ycat, ycat,
      dis, b2, g2, bb2, W3)


def _k4_body(a0, a1, a2, a3, y0, y1, y2, y3, dis_ref, b3_ref, g3_ref,
             bb3_ref, batch_ref, bs_ref, wf_ref, bf_ref, out_ref, psum_ref):
    r = pl.program_id(0)
    nsteps = pl.num_programs(0)

    @pl.when(r == 0)
    def _():
        psum_ref[...] = jnp.zeros_like(psum_ref)

    dis = dis_ref[...]
    acc = jnp.concatenate([a0[0], a1[0], a2[0], a3[0]], axis=-1)
    y = jnp.concatenate([y0[0], y1[0], y2[0], y3[0]], axis=-1)
    pre = dis * (acc + y) + b3_ref[...]
    h = _ln(pre, g3_ref[...], bb3_ref[...])
    hx = jnp.concatenate([h, jnp.ones((h.shape[0], 128), jnp.float32)], axis=-1)
    cols = lax.broadcasted_iota(jnp.int32, (h.shape[0], 128), 1)
    P = (batch_ref[...] == cols).astype(jnp.float32)
    psum_ref[...] += lax.dot_general(
        P, hx, (((0,), (0,)), ((), ())), preferred_element_type=jnp.float32
    )

    @pl.when(r == nsteps - 1)
    def _():
        ps = psum_ref[...]
        cnt = jnp.maximum(ps[:, 256:257], 1.0)
        pooled = ps[:, :256] / cnt
        valid = lax.broadcasted_iota(jnp.int32, (128, 1), 0) < bs_ref[0]
        pooled = jnp.where(valid, pooled, 0.0)
        out_ref[...] = (
            jnp.dot(pooled, wf_ref[...], preferred_element_type=jnp.float32)
            + bf_ref[...]
        )


def _tc_k4(accs, ycat, dis, b3, g3, bb3, batch2d, bs, Wf, bf):
    grid = (N // ROWS_BLK,)
    return pl.pallas_call(
        _k4_body,
        grid=grid,
        in_specs=_chunk_specs(4) + _chunk_specs(4) + [
            pl.BlockSpec((ROWS_BLK, 1), lambda r: (r, 0)),
            pl.BlockSpec((1, 256), lambda r: (0, 0)),
            pl.BlockSpec((1, 256), lambda r: (0, 0)),
            pl.BlockSpec((1, 256), lambda r: (0, 0)),
            pl.BlockSpec((ROWS_BLK, 1), lambda r: (r, 0)),
            pl.BlockSpec(memory_space=pltpu.SMEM),
            pl.BlockSpec((256, 128), lambda r: (0, 0)),
            pl.BlockSpec((1, 128), lambda r: (0, 0)),
        ],
        out_specs=pl.BlockSpec((128, 128), lambda r: (0, 0)),
        out_shape=jax.ShapeDtypeStruct((128, 128), jnp.float32),
        scratch_shapes=[pltpu.VMEM((128, 384), jnp.float32)],
    )(accs, accs, accs, accs, ycat, ycat, ycat, ycat,
      dis, b3, g3, bb3, batch2d, bs, Wf, bf)


def kernel(x, edge_index, batch, batch_size, W1, b1, ln1_g, ln1_b, W2, b2,
           ln2_g, ln2_b, W3, b3, ln3_g, ln3_b, Wf, bf):
    row = edge_index[0].astype(jnp.int32)
    col = edge_index[1].astype(jnp.int32)
    pad = EP - E
    rows_g = jnp.concatenate(
        [row, jnp.zeros((pad,), jnp.int32)]).reshape(NG, GE)
    cols_g = jnp.concatenate(
        [col, jnp.full((pad,), N, jnp.int32)]).reshape(NG, GE)

    zeros16 = jnp.zeros((128, 16), jnp.float32)
    ones16 = jnp.ones((128, 16), jnp.float32)
    zeros64 = jnp.zeros((128, 64), jnp.float32)

    degp = _deg_kernel(cols_g, zeros16, ones16)
    d0 = degp[0, :N, 0:1]
    d1 = degp[1, :N, 0:1]
    y1, dis = _tc_k1(x, d0, d1, W1)

    acc1 = _msg_1pass(y1[0], y1[1], y1[0], y1[1], rows_g, cols_g, zeros64)
    y2cat = _tc_k2(acc1, y1, dis, b1.reshape(1, -1),
                   ln1_g.reshape(1, -1), ln1_b.reshape(1, -1), W2)

    acc2 = _msg_2pass(y2cat[0], y2cat[1], y2cat[2], y2cat[3],
                      rows_g, cols_g, zeros64)
    y3cat = _tc_k3(acc2, y2cat, dis, b2.reshape(1, -1),
                   ln2_g.reshape(1, -1), ln2_b.reshape(1, -1), W3)

    acc3 = _msg_2pass(y3cat[0], y3cat[1], y3cat[2], y3cat[3],
                      rows_g, cols_g, zeros64)

    bs = jnp.asarray(batch_size, jnp.int32).reshape(1)
    out = _tc_k4(acc3, y3cat, dis, b3.reshape(1, -1),
                 ln3_g.reshape(1, -1), ln3_b.reshape(1, -1),
                 batch.astype(jnp.int32).reshape(N, 1), bs, Wf,
                 bf.reshape(1, -1))
    return out


# speedup vs baseline: 1.7503x; 1.6578x over previous
import functools

import jax
import jax.numpy as jnp
from jax import lax
from jax.experimental import pallas as pl
from jax.experimental.pallas import tpu as pltpu
from jax.experimental.pallas import tpu_sc as plsc

N = 10000
E = 320000
NPAD = 10240
EP = 327680
NG = EP // 128
G = 2
TR = NPAD // 16
ROWS_BLK = 2000

_sc_mesh = functools.partial(
    plsc.VectorSubcoreMesh, core_axis_name="c", subcore_axis_name="s"
)


@functools.partial(
    pl.kernel,
    out_type=jax.ShapeDtypeStruct((2, NPAD, 16), jnp.float32),
    mesh=_sc_mesh(),
    scratch_types=[
        pltpu.VMEM_SHARED((NPAD, 16), jnp.float32),
        pltpu.VMEM((G, 128), jnp.int32),
        pltpu.VMEM((128, 16), jnp.float32),
    ],
)
def _deg_kernel(cols_hbm, zeros16, ones16, out, acc, cidx, obuf):
    c = lax.axis_index("c")
    s = lax.axis_index("s")
    wid = c * 16 + s
    pltpu.sync_copy(ones16, obuf)
    for k in range(TR // 128):
        pltpu.sync_copy(zeros16, acc.at[pl.ds(s * TR + k * 128, 128)])
    plsc.subcore_barrier()

    tile_g = NG // 32
    g0 = wid * tile_g

    def body(i, carry):
        gb = g0 + i * G
        pltpu.sync_copy(cols_hbm.at[pl.ds(gb, G)], cidx)
        for g in range(G):
            pltpu.sync_copy(obuf, acc.at[cidx.at[g]], add=True)
        return carry

    lax.fori_loop(0, tile_g // G, body, 0)
    plsc.subcore_barrier()
    pltpu.sync_copy(acc.at[pl.ds(s * TR, TR)], out.at[c].at[pl.ds(s * TR, TR)])


GE = 128
NB = 8
IG = 8
NSET = 3
TILE_G = NG // 16
NBATCH = TILE_G // IG


def _make_msg_kernel(npass):
    @functools.partial(
        pl.kernel,
        out_type=jax.ShapeDtypeStruct((2 * npass, NPAD, 64), jnp.bfloat16),
        mesh=_sc_mesh(),
        scratch_types=[
            pltpu.VMEM_SHARED((NPAD, 64), jnp.bfloat16),
            pltpu.VMEM((NSET, IG, GE), jnp.int32),
            pltpu.VMEM((NSET, IG, GE), jnp.int32),
            pltpu.VMEM((NB, GE, 64), jnp.bfloat16),
            [pltpu.SemaphoreType.DMA] * NB,
            [pltpu.SemaphoreType.DMA] * NB,
            pltpu.SemaphoreType.DMA,
        ],
        compiler_params=pltpu.CompilerParams(use_tc_tiling_on_sc=False),
    )
    def k(y0, y1, y2, y3, rows_hbm, cols_hbm, zeros64, out, acc, ridx, cidx,
          dbuf, semg, sems, semi):
        c = lax.axis_index("c")
        s = lax.axis_index("s")
        ys = [y0, y1, y2, y3]
        g0 = s * TILE_G

        def stage_idx(set_, batch):
            gb = g0 + batch * IG
            pltpu.async_copy(rows_hbm.at[pl.ds(gb, IG)], ridx.at[set_], semi)
            pltpu.async_copy(cols_hbm.at[pl.ds(gb, IG)], cidx.at[set_], semi)

        def wait_idx(set_):
            pltpu.make_async_copy(
                rows_hbm.at[pl.ds(g0, IG)], ridx.at[set_], semi).wait()
            pltpu.make_async_copy(
                cols_hbm.at[pl.ds(g0, IG)], cidx.at[set_], semi).wait()

        for t in range(npass):
            src0 = ys[t]
            src1 = ys[npass + t]

            def fire_gather(idx_slice, b):
                @pl.when(c == 0)
                def _():
                    pltpu.async_copy(src0.at[idx_slice], dbuf.at[b], semg[b])

                @pl.when(c == 1)
                def _():
                    pltpu.async_copy(src1.at[idx_slice], dbuf.at[b], semg[b])

            def wait_gather(idx_slice, b):
                pltpu.make_async_copy(
                    src0.at[idx_slice], dbuf.at[b], semg[b]).wait()

            def fire_scatter(b, idx_slice):
                pltpu.async_copy(dbuf.at[b], acc.at[idx_slice], sems[b],
                                 add=True)

            def wait_scatter(b, idx_slice):
                pltpu.make_async_copy(
                    dbuf.at[b], acc.at[idx_slice], sems[b]).wait()

            for k_ in range(TR // 128):
                pltpu.sync_copy(zeros64, acc.at[pl.ds(s * TR + k_ * 128, 128)])
            plsc.subcore_barrier()

            def batch_steps(p, pn, first, last, stage_fn=None):
                for j in range(IG):
                    b = j % NB
                    wait_gather(ridx.at[p, j], b)
                    fire_scatter(b, cidx.at[p, j])
                    bprev = (j - 1) % NB
                    if j == 0:
                        if not first:
                            wait_scatter(bprev, cidx.at[p, IG - 1])
                        fire_gather(ridx.at[p, IG - 1], IG - 1)
                    else:
                        wait_scatter(bprev, cidx.at[p, j - 1])
                        if j == 1 and not last:
                            wait_idx(pn)
                        if j == 6 and stage_fn is not None:
                            stage_fn()
                        if not last:
                            fire_gather(ridx.at[pn, j - 1], bprev)

            stage_idx(0, 0)
            wait_idx(0)
            stage_idx(1, 1)
            for j in range(NB - 1):
                fire_gather(ridx.at[0, j], j)

            batch_steps(0, 1, first=True, last=False,
                        stage_fn=lambda: stage_idx(2, 2))

            def body(kb, carry):
                p = lax.rem(kb, NSET)
                pn = lax.rem(kb + 1, NSET)

                def stage():
                    @pl.when(kb + 2 < NBATCH)
                    def _():
                        stage_idx(lax.rem(kb + 2, NSET), kb + 2)

                batch_steps(p, pn, first=False, last=False, stage_fn=stage)
                return carry

            lax.fori_loop(1, NBATCH - 1, body, 0)

            pf = (NBATCH - 1) % NSET
            batch_steps(pf, 0, first=False, last=True)
            wait_scatter(IG - 1, cidx.at[pf, IG - 1])

            plsc.subcore_barrier()
            pltpu.sync_copy(
                acc.at[pl.ds(s * TR, TR)],
                out.at[c * npass + t].at[pl.ds(s * TR, TR)],
            )
            if t + 1 < npass:
                plsc.subcore_barrier()

    return k


_msg_1pass = _make_msg_kernel(1)
_msg_2pass = _make_msg_kernel(2)


def _split64(y_ref, y):
    for i in range(y_ref.shape[0]):
        y_ref[i] = y[:, 64 * i:64 * i + 64]


def _k1_body(x_ref, d0_ref, d1_ref, w1_ref, y_ref, dis_ref):
    deg = d0_ref[...] + d1_ref[...] + 1.0
    dis = lax.rsqrt(deg)
    dis_ref[...] = dis
    _split64(y_ref, (jnp.dot(x_ref[...], w1_ref[...],
                             preferred_element_type=jnp.float32)
                     * dis).astype(jnp.bfloat16))


def _tc_k1(x, d0, d1, W1):
    grid = (N // ROWS_BLK,)
    return pl.pallas_call(
        _k1_body,
        grid=grid,
        in_specs=[
            pl.BlockSpec((ROWS_BLK, 128), lambda r: (r, 0)),
            pl.BlockSpec((ROWS_BLK, 1), lambda r: (r, 0)),
            pl.BlockSpec((ROWS_BLK, 1), lambda r: (r, 0)),
            pl.BlockSpec((128, 128), lambda r: (0, 0)),
        ],
        out_specs=[
            pl.BlockSpec((2, ROWS_BLK, 64), lambda r: (0, r, 0)),
            pl.BlockSpec((ROWS_BLK, 1), lambda r: (r, 0)),
        ],
        out_shape=[
            jax.ShapeDtypeStruct((2, N, 64), jnp.bfloat16),
            jax.ShapeDtypeStruct((N, 1), jnp.float32),
        ],
    )(x, d0, d1, W1)


def _ln(pre, g, b):
    m = jnp.mean(pre, axis=-1, keepdims=True)
    v = jnp.mean((pre - m) * (pre - m), axis=-1, keepdims=True)
    return (pre - m) * lax.rsqrt(v + 1e-5) * g + b


def _leaky(h):
    return jnp.where(h >= 0, h, 0.01 * h)


def _k2_body(a0_ref, a1_ref, y1a_ref, y1b_ref, dis_ref, b1_ref, g1_ref,
             bb1_ref, w2_ref, y2_ref):
    dis = dis_ref[...]
    acc = jnp.concatenate([a0_ref[0], a1_ref[0]], axis=-1).astype(jnp.float32)
    y1 = jnp.concatenate([y1a_ref[0], y1b_ref[0]], axis=-1).astype(jnp.float32)
    pre = dis * (acc + y1) + b1_ref[...]
    h = _leaky(_ln(pre, g1_ref[...], bb1_ref[...]))
    _split64(y2_ref, (jnp.dot(h, w2_ref[...],
                              preferred_element_type=jnp.float32)
                      * dis).astype(jnp.bfloat16))


def _tc_k2(accs, y1, dis, b1, g1, bb1, W2):
    grid = (N // ROWS_BLK,)
    return pl.pallas_call(
        _k2_body,
        grid=grid,
        in_specs=[
            pl.BlockSpec((1, ROWS_BLK, 64), lambda r: (0, r, 0)),
            pl.BlockSpec((1, ROWS_BLK, 64), lambda r: (1, r, 0)),
            pl.BlockSpec((1, ROWS_BLK, 64), lambda r: (0, r, 0)),
            pl.BlockSpec((1, ROWS_BLK, 64), lambda r: (1, r, 0)),
            pl.BlockSpec((ROWS_BLK, 1), lambda r: (r, 0)),
            pl.BlockSpec((1, 128), lambda r: (0, 0)),
            pl.BlockSpec((1, 128), lambda r: (0, 0)),
            pl.BlockSpec((1, 128), lambda r: (0, 0)),
            pl.BlockSpec((128, 256), lambda r: (0, 0)),
        ],
        out_specs=pl.BlockSpec((4, ROWS_BLK, 64), lambda r: (0, r, 0)),
        out_shape=jax.ShapeDtypeStruct((4, N, 64), jnp.bfloat16),
    )(accs, accs, y1, y1, dis, b1, g1, bb1, W2)


def _chunk_specs(n):
    return [pl.BlockSpec((1, ROWS_BLK, 64), (lambda i: lambda r: (i, r, 0))(i))
            for i in range(n)]


def _k3_body(a0, a1, a2, a3, y0, y1, y2, y3, dis_ref, b2_ref, g2_ref,
             bb2_ref, w3_ref, y3_ref):
    dis = dis_ref[...]
    acc = jnp.concatenate([a0[0], a1[0], a2[0], a3[0]],
                          axis=-1).astype(jnp.float32)
    y = jnp.concatenate([y0[0], y1[0], y2[0], y3[0]],
                        axis=-1).astype(jnp.float32)
    pre = dis * (acc + y) + b2_ref[...]
    h = _leaky(_ln(pre, g2_ref[...], bb2_ref[...]))
    _split64(y3_ref, (jnp.dot(h, w3_ref[...],
                              preferred_element_type=jnp.float32)
                      * dis).astype(jnp.bfloat16))


def _tc_k3(accs, ycat, dis, b2, g2, bb2, W3):
    grid = (N // ROWS_BLK,)
    return pl.pallas_call(
        _k3_body,
        grid=grid,
        in_specs=_chunk_specs(4) + _chunk_specs(4) + [
            pl.BlockSpec((ROWS_BLK, 1), lambda r: (r, 0)),
            pl.BlockSpec((1, 256), lambda r: (0, 0)),
            pl.BlockSpec((1, 256), lambda r: (0, 0)),
            pl.BlockSpec((1, 256), lambda r: (0, 0)),
            pl.BlockSpec((256, 256), lambda r: (0, 0)),
        ],
        out_specs=pl.BlockSpec((4, ROWS_BLK, 64), lambda r: (0, r, 0)),
        out_shape=jax.ShapeDtypeStruct((4, N, 64), jnp.bfloat16),
    )(accs, accs, accs, accs, ycat, ycat, ycat, ycat,
      dis, b2, g2, bb2, W3)


def _k4_body(a0, a1, a2, a3, y0, y1, y2, y3, dis_ref, b3_ref, g3_ref,
             bb3_ref, batch_ref, bs_ref, wf_ref, bf_ref, out_ref, psum_ref):
    r = pl.program_id(0)
    nsteps = pl.num_programs(0)

    @pl.when(r == 0)
    def _():
        psum_ref[...] = jnp.zeros_like(psum_ref)

    dis = dis_ref[...]
    acc = jnp.concatenate([a0[0], a1[0], a2[0], a3[0]],
                          axis=-1).astype(jnp.float32)
    y = jnp.concatenate([y0[0], y1[0], y2[0], y3[0]],
                        axis=-1).astype(jnp.float32)
    pre = dis * (acc + y) + b3_ref[...]
    h = _ln(pre, g3_ref[...], bb3_ref[...])
    hx = jnp.concatenate([h, jnp.ones((h.shape[0], 128), jnp.float32)], axis=-1)
    cols = lax.broadcasted_iota(jnp.int32, (h.shape[0], 128), 1)
    P = (batch_ref[...] == cols).astype(jnp.float32)
    psum_ref[...] += lax.dot_general(
        P, hx, (((0,), (0,)), ((), ())), preferred_element_type=jnp.float32
    )

    @pl.when(r == nsteps - 1)
    def _():
        ps = psum_ref[...]
        cnt = jnp.maximum(ps[:, 256:257], 1.0)
        pooled = ps[:, :256] / cnt
        valid = lax.broadcasted_iota(jnp.int32, (128, 1), 0) < bs_ref[0]
        pooled = jnp.where(valid, pooled, 0.0)
        out_ref[...] = (
            jnp.dot(pooled, wf_ref[...], preferred_element_type=jnp.float32)
            + bf_ref[...]
        )


def _tc_k4(accs, ycat, dis, b3, g3, bb3, batch2d, bs, Wf, bf):
    grid = (N // ROWS_BLK,)
    return pl.pallas_call(
        _k4_body,
        grid=grid,
        in_specs=_chunk_specs(4) + _chunk_specs(4) + [
            pl.BlockSpec((ROWS_BLK, 1), lambda r: (r, 0)),
            pl.BlockSpec((1, 256), lambda r: (0, 0)),
            pl.BlockSpec((1, 256), lambda r: (0, 0)),
            pl.BlockSpec((1, 256), lambda r: (0, 0)),
            pl.BlockSpec((ROWS_BLK, 1), lambda r: (r, 0)),
            pl.BlockSpec(memory_space=pltpu.SMEM),
            pl.BlockSpec((256, 128), lambda r: (0, 0)),
            pl.BlockSpec((1, 128), lambda r: (0, 0)),
        ],
        out_specs=pl.BlockSpec((128, 128), lambda r: (0, 0)),
        out_shape=jax.ShapeDtypeStruct((128, 128), jnp.float32),
        scratch_shapes=[pltpu.VMEM((128, 384), jnp.float32)],
    )(accs, accs, accs, accs, ycat, ycat, ycat, ycat,
      dis, b3, g3, bb3, batch2d, bs, Wf, bf)


def kernel(x, edge_index, batch, batch_size, W1, b1, ln1_g, ln1_b, W2, b2,
           ln2_g, ln2_b, W3, b3, ln3_g, ln3_b, Wf, bf):
    row = edge_index[0].astype(jnp.int32)
    col = edge_index[1].astype(jnp.int32)
    pad = EP - E
    rows_g = jnp.concatenate(
        [row, jnp.zeros((pad,), jnp.int32)]).reshape(NG, GE)
    cols_g = jnp.concatenate(
        [col, jnp.full((pad,), N, jnp.int32)]).reshape(NG, GE)

    zeros16 = jnp.zeros((128, 16), jnp.float32)
    ones16 = jnp.ones((128, 16), jnp.float32)
    zeros64 = jnp.zeros((128, 64), jnp.bfloat16)

    degp = _deg_kernel(cols_g, zeros16, ones16)
    d0 = degp[0, :N, 0:1]
    d1 = degp[1, :N, 0:1]
    y1, dis = _tc_k1(x, d0, d1, W1)

    acc1 = _msg_1pass(y1[0], y1[1], y1[0], y1[1], rows_g, cols_g, zeros64)
    y2cat = _tc_k2(acc1, y1, dis, b1.reshape(1, -1),
                   ln1_g.reshape(1, -1), ln1_b.reshape(1, -1), W2)

    acc2 = _msg_2pass(y2cat[0], y2cat[1], y2cat[2], y2cat[3],
                      rows_g, cols_g, zeros64)
    y3cat = _tc_k3(acc2, y2cat, dis, b2.reshape(1, -1),
                   ln2_g.reshape(1, -1), ln2_b.reshape(1, -1), W3)

    acc3 = _msg_2pass(y3cat[0], y3cat[1], y3cat[2], y3cat[3],
                      rows_g, cols_g, zeros64)

    bs = jnp.asarray(batch_size, jnp.int32).reshape(1)
    out = _tc_k4(acc3, y3cat, dis, b3.reshape(1, -1),
                 ln3_g.reshape(1, -1), ln3_b.reshape(1, -1),
                 batch.astype(jnp.int32).reshape(N, 1), bs, Wf,
                 bf.reshape(1, -1))
    return out
